# initial kernel scaffold (unmeasured)
import jax
import jax.numpy as jnp
from jax import lax
from jax.experimental import pallas as pl
from jax.experimental.pallas import tpu as pltpu


def kernel(
    x,
):
    def body(*refs):
        pass

    out_shape = jax.ShapeDtypeStruct(..., jnp.float32)
    return pl.pallas_call(body, out_shape=out_shape)(...)


# baseline (device time: 75911 ns/iter reference)
import jax
import jax.numpy as jnp
from jax import lax
from jax.experimental import pallas as pl
from jax.experimental.pallas import tpu as pltpu

N_DEV = 4


def kernel(x):
    m, n = x.shape
    TN = 256
    nt = n // TN

    def body(x_ref, o_ref, own_ref, tot_ref, ssems, rsems):
        i = pl.program_id(0)
        my = lax.axis_index("i")

        barrier = pltpu.get_barrier_semaphore()

        @pl.when(i == 0)
        def _():
            for d in range(1, N_DEV):
                pl.semaphore_signal(
                    barrier,
                    inc=1,
                    device_id=((my + d) % N_DEV,),
                    device_id_type=pl.DeviceIdType.MESH,
                )
            pl.semaphore_wait(barrier, N_DEV - 1)

        xv = x_ref[...]

        tot = xv
        h = m // 2
        while h >= 1:
            tot = tot[:h] * tot[h : 2 * h]
            h //= 2
        own_ref[pl.ds(i, 1), :] = tot

        rdmas = []
        for d in range(1, N_DEV):
            rdma = pltpu.make_async_remote_copy(
                src_ref=own_ref.at[pl.ds(i, 1)],
                dst_ref=tot_ref.at[pl.ds(i * (N_DEV - 1) + d - 1, 1)],
                send_sem=ssems.at[i * (N_DEV - 1) + d - 1],
                recv_sem=rsems.at[i * (N_DEV - 1) + d - 1],
                device_id=((my + d) % N_DEV,),
                device_id_type=pl.DeviceIdType.MESH,
            )
            rdma.start()
            rdmas.append(rdma)

        y = xv
        k = 1
        while k < m:
            shifted = jnp.concatenate(
                [jnp.ones((k, TN), jnp.float32), y[: m - k]], axis=0
            )
            y = y * shifted
            k *= 2

        pref = jnp.ones((1, TN), jnp.float32)
        for d in range(1, N_DEV):
            rdmas[d - 1].wait_recv()
            src_dev = (my - d) % N_DEV
            tvals = tot_ref[pl.ds(i * (N_DEV - 1) + d - 1, 1), :]
            pref = pref * jnp.where(src_dev < my, tvals, jnp.ones_like(tvals))
        for d in range(1, N_DEV):
            rdmas[d - 1].wait_send()

        o_ref[...] = (pref * y).astype(o_ref.dtype)

    return pl.pallas_call(
        body,
        grid=(nt,),
        out_shape=jax.ShapeDtypeStruct((m, n), jnp.bfloat16),
        in_specs=[pl.BlockSpec((m, TN), lambda i: (0, i))],
        out_specs=pl.BlockSpec((m, TN), lambda i: (0, i)),
        scratch_shapes=[
            pltpu.VMEM((nt, TN), jnp.float32),
            pltpu.VMEM((nt * (N_DEV - 1), TN), jnp.float32),
            pltpu.SemaphoreType.DMA((nt * (N_DEV - 1),)),
            pltpu.SemaphoreType.DMA((nt * (N_DEV - 1),)),
        ],
        compiler_params=pltpu.CompilerParams(
            dimension_semantics=("arbitrary",),
            collective_id=0,
            vmem_limit_bytes=100 * 1024 * 1024,
        ),
    )(x)


# device time: 61225 ns/iter; 1.2399x vs baseline; 1.2399x over previous
import jax
import jax.numpy as jnp
from jax import lax
from jax.experimental import pallas as pl
from jax.experimental.pallas import tpu as pltpu

N_DEV = 4


def kernel(x):
    m, n = x.shape
    TN = 256
    nt = n // TN

    def body(x_ref, o_ref, own_ref, tot_ref, ssems, rsems):
        i = pl.program_id(0)
        my = lax.axis_index("i")

        barrier = pltpu.get_barrier_semaphore()

        @pl.when(i == 0)
        def _():
            for d in range(1, N_DEV):
                pl.semaphore_signal(
                    barrier,
                    inc=1,
                    device_id=((my + d) % N_DEV,),
                    device_id_type=pl.DeviceIdType.MESH,
                )
            pl.semaphore_wait(barrier, N_DEV - 1)

        xv = x_ref[...]

        C, R = m // 64, 64
        a = xv.reshape(C, R, TN)
        k = 1
        while k < R:
            shifted = jnp.concatenate(
                [jnp.ones((C, k, TN), jnp.float32), a[:, : R - k, :]], axis=1
            )
            a = a * shifted
            k *= 2

        p = a[:, R - 1 : R, :].reshape(C, TN)
        inc = p
        k = 1
        while k < C:
            shifted = jnp.concatenate(
                [jnp.ones((k, TN), jnp.float32), inc[: C - k]], axis=0
            )
            inc = inc * shifted
            k *= 2

        own_ref[pl.ds(i, 1), :] = inc[C - 1 : C, :]
        rdmas = []
        for d in range(1, N_DEV):
            rdma = pltpu.make_async_remote_copy(
                src_ref=own_ref.at[pl.ds(i, 1)],
                dst_ref=tot_ref.at[pl.ds(i * (N_DEV - 1) + d - 1, 1)],
                send_sem=ssems.at[i * (N_DEV - 1) + d - 1],
                recv_sem=rsems.at[i * (N_DEV - 1) + d - 1],
                device_id=((my + d) % N_DEV,),
                device_id_type=pl.DeviceIdType.MESH,
            )
            rdma.start()
            rdmas.append(rdma)

        exc = jnp.concatenate(
            [jnp.ones((1, TN), jnp.float32), inc[: C - 1]], axis=0
        )

        pref = jnp.ones((1, TN), jnp.float32)
        for d in range(1, N_DEV):
            rdmas[d - 1].wait_recv()
            src_dev = (my - d) % N_DEV
            tvals = tot_ref[pl.ds(i * (N_DEV - 1) + d - 1, 1), :]
            pref = pref * jnp.where(src_dev < my, tvals, jnp.ones_like(tvals))
        for d in range(1, N_DEV):
            rdmas[d - 1].wait_send()

        scale = exc * pref
        out = a * scale[:, None, :]
        o_ref[...] = out.reshape(m, TN).astype(o_ref.dtype)

    return pl.pallas_call(
        body,
        grid=(nt,),
        out_shape=jax.ShapeDtypeStruct((m, n), jnp.bfloat16),
        in_specs=[pl.BlockSpec((m, TN), lambda i: (0, i))],
        out_specs=pl.BlockSpec((m, TN), lambda i: (0, i)),
        scratch_shapes=[
            pltpu.VMEM((nt, TN), jnp.float32),
            pltpu.VMEM((nt * (N_DEV - 1), TN), jnp.float32),
            pltpu.SemaphoreType.DMA((nt * (N_DEV - 1),)),
            pltpu.SemaphoreType.DMA((nt * (N_DEV - 1),)),
        ],
        compiler_params=pltpu.CompilerParams(
            dimension_semantics=("arbitrary",),
            collective_id=0,
            vmem_limit_bytes=100 * 1024 * 1024,
        ),
    )(x)


# device time: 41988 ns/iter; 1.8079x vs baseline; 1.4582x over previous
import jax
import jax.numpy as jnp
from jax import lax
from jax.experimental import pallas as pl
from jax.experimental.pallas import tpu as pltpu

N_DEV = 4


def kernel(x):
    m, n = x.shape
    TN = 256
    nt = n // TN

    def body(x_ref, o_ref, own_ref, tot_ref, ssems, rsems):
        i = pl.program_id(0)
        my = lax.axis_index("i")

        barrier = pltpu.get_barrier_semaphore()

        @pl.when(i == 0)
        def _():
            for d in range(1, N_DEV):
                pl.semaphore_signal(
                    barrier,
                    inc=1,
                    device_id=((my + d) % N_DEV,),
                    device_id_type=pl.DeviceIdType.MESH,
                )
            pl.semaphore_wait(barrier, N_DEV - 1)

        xv = x_ref[...]

        C, R = m // 64, 64
        s = jnp.log(xv).reshape(C, R, TN)
        row_i = lax.broadcasted_iota(jnp.int32, (R, R), 0)
        col_i = lax.broadcasted_iota(jnp.int32, (R, R), 1)
        ltri = (row_i >= col_i).astype(jnp.float32)
        ltri_b = jnp.broadcast_to(ltri, (C, R, R))
        S = lax.dot_general(
            ltri_b,
            s,
            dimension_numbers=(((2,), (1,)), ((0,), (0,))),
            preferred_element_type=jnp.float32,
        )

        cs = S[:, R - 1 : R, :].reshape(C, TN)
        row_c = lax.broadcasted_iota(jnp.int32, (C, C), 0)
        col_c = lax.broadcasted_iota(jnp.int32, (C, C), 1)
        inc = lax.dot_general(
            (row_c >= col_c).astype(jnp.float32),
            cs,
            dimension_numbers=(((1,), (0,)), ((), ())),
            preferred_element_type=jnp.float32,
        )

        own_ref[pl.ds(i, 1), :] = inc[C - 1 : C, :]
        rdmas = []
        for d in range(1, N_DEV):
            rdma = pltpu.make_async_remote_copy(
                src_ref=own_ref.at[pl.ds(i, 1)],
                dst_ref=tot_ref.at[pl.ds(i * (N_DEV - 1) + d - 1, 1)],
                send_sem=ssems.at[i * (N_DEV - 1) + d - 1],
                recv_sem=rsems.at[i * (N_DEV - 1) + d - 1],
                device_id=((my + d) % N_DEV,),
                device_id_type=pl.DeviceIdType.MESH,
            )
            rdma.start()
            rdmas.append(rdma)

        exc = jnp.concatenate(
            [jnp.zeros((1, TN), jnp.float32), inc[: C - 1]], axis=0
        )

        pref = jnp.zeros((1, TN), jnp.float32)
        for d in range(1, N_DEV):
            rdmas[d - 1].wait_recv()
            src_dev = (my - d) % N_DEV
            tvals = tot_ref[pl.ds(i * (N_DEV - 1) + d - 1, 1), :]
            pref = pref + jnp.where(src_dev < my, tvals, jnp.zeros_like(tvals))
        for d in range(1, N_DEV):
            rdmas[d - 1].wait_send()

        offs = exc + pref
        out = jnp.exp(S + offs[:, None, :])
        o_ref[...] = out.reshape(m, TN).astype(o_ref.dtype)

    return pl.pallas_call(
        body,
        grid=(nt,),
        out_shape=jax.ShapeDtypeStruct((m, n), jnp.bfloat16),
        in_specs=[pl.BlockSpec((m, TN), lambda i: (0, i))],
        out_specs=pl.BlockSpec((m, TN), lambda i: (0, i)),
        scratch_shapes=[
            pltpu.VMEM((nt, TN), jnp.float32),
            pltpu.VMEM((nt * (N_DEV - 1), TN), jnp.float32),
            pltpu.SemaphoreType.DMA((nt * (N_DEV - 1),)),
            pltpu.SemaphoreType.DMA((nt * (N_DEV - 1),)),
        ],
        compiler_params=pltpu.CompilerParams(
            dimension_semantics=("arbitrary",),
            collective_id=0,
            vmem_limit_bytes=100 * 1024 * 1024,
        ),
    )(x)
